# Initial kernel scaffold; baseline (speedup 1.0000x reference)
import jax
import jax.numpy as jnp
from jax.experimental import pallas as pl


def kernel(x, edge_index, edge_attr, W_msg, b_msg, W_node, b_node, W1, b1, W2, b2, W3, b3):
    raise NotImplementedError("write your pallas kernel here")


# speedup vs baseline: 3.4618x; 3.4618x over previous
import functools

import jax
import jax.numpy as jnp
from jax import lax
from jax.experimental import pallas as pl
from jax.experimental.pallas import tpu as pltpu
from jax.experimental.pallas import tpu_sc as plsc

_NC = 2
_NS = 16
_CH = 128


def _sc_edge_aggregate(xs2, src2, dst, ea, zg, za, ones_blk,
                       n_pad, n_chunks, iters):
    dh = xs2.shape[1]
    de = ea.shape[1]
    rows_per_tile = n_pad // _NS
    mesh = plsc.VectorSubcoreMesh(core_axis_name="c", subcore_axis_name="s",
                                  num_cores=_NC)

    @functools.partial(
        pl.kernel,
        mesh=mesh,
        compiler_params=pltpu.CompilerParams(use_tc_tiling_on_sc=False),
        out_type=(
            jax.ShapeDtypeStruct((_NC, n_pad, dh), jnp.float32),
            jax.ShapeDtypeStruct((_NC, n_pad, de), jnp.float32),
            jax.ShapeDtypeStruct((_NC, n_pad, de), jnp.float32),
        ),
        scratch_types=(
            pltpu.VMEM((_CH,), jnp.int32),
            pltpu.VMEM((_CH,), jnp.int32),
            pltpu.VMEM((_CH, dh), jnp.float32),
            pltpu.VMEM((_CH, de), jnp.float32),
            pltpu.VMEM((_CH, de), jnp.float32),
            pltpu.VMEM_SHARED((n_pad, dh), jnp.float32),
            pltpu.VMEM_SHARED((n_pad, de), jnp.float32),
            pltpu.VMEM_SHARED((n_pad, de), jnp.float32),
            pltpu.SemaphoreType.DMA,
        ),
    )
    def k(xs2_hbm, src2_hbm, dst_hbm, ea_hbm, zg_hbm, za_hbm, ones_hbm,
          g_out, a_out, c_out,
          src_v, dst_v, xrows_v, ea_v, ones_v, g_sh, a_sh, c_sh, sem):
        c = lax.axis_index("c")
        s = lax.axis_index("s")
        r0 = s * rows_per_tile
        src_off = c * (n_chunks * _CH)

        pltpu.sync_copy(zg_hbm.at[pl.ds(r0, rows_per_tile)],
                        g_sh.at[pl.ds(r0, rows_per_tile)])
        pltpu.sync_copy(za_hbm.at[pl.ds(r0, rows_per_tile)],
                        a_sh.at[pl.ds(r0, rows_per_tile)])
        pltpu.sync_copy(za_hbm.at[pl.ds(r0, rows_per_tile)],
                        c_sh.at[pl.ds(r0, rows_per_tile)])
        pltpu.sync_copy(ones_hbm, ones_v)
        plsc.subcore_barrier()

        def body(kk, carry):
            cid = s + kk * _NS

            @pl.when(cid < n_chunks)
            def _():
                base = cid * _CH
                pltpu.sync_copy(src2_hbm.at[pl.ds(src_off + base, _CH)], src_v)
                pltpu.sync_copy(dst_hbm.at[pl.ds(base, _CH)], dst_v)
                pltpu.async_copy(xs2_hbm.at[src_v], xrows_v, sem).wait()
                pltpu.sync_copy(xrows_v, g_sh.at[dst_v], add=True)

                @pl.when(cid % _NC == c)
                def _():
                    pltpu.sync_copy(ea_hbm.at[pl.ds(base, _CH)], ea_v)
                    pltpu.sync_copy(ea_v, a_sh.at[dst_v], add=True)
                    pltpu.sync_copy(ones_v, c_sh.at[dst_v], add=True)

            return carry

        lax.fori_loop(0, iters, body, 0)
        plsc.subcore_barrier()

        pltpu.sync_copy(g_sh.at[pl.ds(r0, rows_per_tile)],
                        g_out.at[c, pl.ds(r0, rows_per_tile)])
        pltpu.sync_copy(a_sh.at[pl.ds(r0, rows_per_tile)],
                        a_out.at[c, pl.ds(r0, rows_per_tile)])
        pltpu.sync_copy(c_sh.at[pl.ds(r0, rows_per_tile)],
                        c_out.at[c, pl.ds(r0, rows_per_tile)])

    return k(xs2, src2, dst, ea, zg, za, ones_blk)


def _tc_dense(g2, a2, c2, xp, wx, we, bm, wn1, wn2, bn, w1, b1, w2, b2, w3, b3):
    n_pad, d = xp.shape
    dh = g2.shape[2]
    de = a2.shape[2]
    d_mlp = w3.shape[1]
    bn_rows = n_pad
    cand8 = n_pad // 8
    for cand in (1280, cand8 if cand8 % 8 == 0 else 0,
                 640, 320, 160, 128, 64, 32, 16, 8):
        if cand <= n_pad and n_pad % cand == 0:
            bn_rows = cand
            break
    grid = (n_pad // bn_rows,)

    def body(g_ref, a_ref, c_ref, x_ref, wx_ref, we_ref, bm_ref,
             wn1_ref, wn2_ref, bn_ref, w1_ref, b1_ref, w2_ref, b2_ref,
             w3_ref, b3_ref, o_ref):
        f32 = jnp.float32
        g = jnp.concatenate([g_ref[i] for i in range(_NC)], axis=-1)
        a = a_ref[0] + a_ref[1]
        cnt = (c_ref[0] + c_ref[1])[:, 0:1]
        summed = (jnp.dot(g, wx_ref[...], preferred_element_type=f32)
                  + jnp.dot(a, we_ref[...], preferred_element_type=f32)
                  + cnt * bm_ref[...])
        agg = summed / jnp.maximum(cnt, 1.0)
        h = (jnp.dot(x_ref[...], wn1_ref[...], preferred_element_type=f32)
             + jnp.dot(agg, wn2_ref[...], preferred_element_type=f32)
             + bn_ref[...])
        h = jnp.maximum(jnp.dot(h, w1_ref[...], preferred_element_type=f32)
                        + b1_ref[...], 0.0)
        h = jnp.maximum(jnp.dot(h, w2_ref[...], preferred_element_type=f32)
                        + b2_ref[...], 0.0)
        o_ref[...] = jnp.dot(h, w3_ref[...], preferred_element_type=f32) + b3_ref[...]

    full = lambda shape: pl.BlockSpec(shape, lambda i: (0,) * len(shape))
    return pl.pallas_call(
        body,
        grid=grid,
        in_specs=[
            pl.BlockSpec((_NC, bn_rows, dh), lambda i: (0, i, 0)),
            pl.BlockSpec((_NC, bn_rows, de), lambda i: (0, i, 0)),
            pl.BlockSpec((_NC, bn_rows, de), lambda i: (0, i, 0)),
            pl.BlockSpec((bn_rows, d), lambda i: (i, 0)),
            full(wx.shape), full(we.shape), full(bm.shape),
            full(wn1.shape), full(wn2.shape), full(bn.shape),
            full(w1.shape), full(b1.shape), full(w2.shape), full(b2.shape),
            full(w3.shape), full(b3.shape),
        ],
        out_specs=pl.BlockSpec((bn_rows, d_mlp), lambda i: (i, 0)),
        out_shape=jax.ShapeDtypeStruct((n_pad, d_mlp), jnp.float32),
    )(g2, a2, c2, xp, wx, we, bm, wn1, wn2, bn, w1, b1, w2, b2, w3, b3)


def kernel(x, edge_index, edge_attr, W_msg, b_msg, W_node, b_node,
           W1, b1, W2, b2, W3, b3):
    n, d = x.shape
    e = edge_index.shape[1]
    de = edge_attr.shape[1]

    rows_per_tile = -(-n // _NS)
    rows_per_tile = -(-rows_per_tile // 8) * 8
    n_pad = rows_per_tile * _NS

    src = edge_index[0].astype(jnp.int32)
    dst = edge_index[1].astype(jnp.int32)
    e_pad = -(-e // _CH) * _CH
    if e_pad != e:
        src = jnp.concatenate([src, jnp.zeros((e_pad - e,), jnp.int32)])
        dst = jnp.concatenate([dst, jnp.full((e_pad - e,), n_pad - 1, jnp.int32)])
        edge_attr = jnp.concatenate(
            [edge_attr, jnp.zeros((e_pad - e, de), jnp.float32)])
    n_chunks = e_pad // _CH
    iters = -(-n_chunks // _NS)

    dh = d // _NC
    xs2 = jnp.concatenate([x[:, c * dh:(c + 1) * dh] for c in range(_NC)])
    src2 = jnp.concatenate([src + c * n for c in range(_NC)])

    zg = jnp.zeros((n_pad, dh), jnp.float32)
    za = jnp.zeros((n_pad, de), jnp.float32)
    ones_blk = jnp.ones((_CH, de), jnp.float32)

    g2, a2, c2 = _sc_edge_aggregate(xs2, src2, dst, edge_attr, zg, za,
                                    ones_blk, n_pad, n_chunks, iters)

    xp = jnp.concatenate([x, jnp.zeros((n_pad - n, d), jnp.float32)])
    wx = W_msg[:, :d].T
    we = W_msg[:, d:].T
    wn1 = W_node[:, :d].T
    wn2 = W_node[:, d:].T
    out = _tc_dense(g2, a2, c2, xp, wx, we, b_msg.reshape(1, -1),
                    wn1, wn2, b_node.reshape(1, -1),
                    W1.T, b1.reshape(1, -1), W2.T, b2.reshape(1, -1),
                    W3.T, b3.reshape(1, -1))
    return out[:n]


# speedup vs baseline: 6.5309x; 1.8866x over previous
import functools

import jax
import jax.numpy as jnp
from jax import lax
from jax.experimental import pallas as pl
from jax.experimental.pallas import tpu as pltpu
from jax.experimental.pallas import tpu_sc as plsc

_NC = 2
_NS = 16
_CH = 128
_GRP = 5


def _sc_edge_aggregate(xs2, src2, dst, ea, zg, za, ones_blk,
                       n_pad, n_groups, iters):
    dh = xs2.shape[1]
    de = ea.shape[1]
    rows_per_tile = n_pad // _NS
    mesh = plsc.VectorSubcoreMesh(core_axis_name="c", subcore_axis_name="s",
                                  num_cores=_NC)

    @functools.partial(
        pl.kernel,
        mesh=mesh,
        compiler_params=pltpu.CompilerParams(use_tc_tiling_on_sc=False),
        out_type=(
            jax.ShapeDtypeStruct((_NC, n_pad, dh), jnp.float32),
            jax.ShapeDtypeStruct((_NC, n_pad, de), jnp.float32),
            jax.ShapeDtypeStruct((_NC, n_pad, de), jnp.float32),
        ),
        scratch_types=(
            pltpu.VMEM((_GRP * _CH,), jnp.int32),
            pltpu.VMEM((_GRP, _CH), jnp.int32),
            pltpu.VMEM((_GRP, _CH, dh), jnp.float32),
            pltpu.VMEM((_GRP, _CH, de), jnp.float32),
            pltpu.VMEM((_CH, de), jnp.float32),
            pltpu.VMEM_SHARED((n_pad, dh), jnp.float32),
            pltpu.VMEM_SHARED((n_pad, de), jnp.float32),
            pltpu.VMEM_SHARED((n_pad, de), jnp.float32),
            pltpu.SemaphoreType.DMA,
            pltpu.SemaphoreType.DMA,
            pltpu.SemaphoreType.DMA,
            pltpu.SemaphoreType.DMA,
        ),
    )
    def k(xs2_hbm, src2_hbm, dst_hbm, ea_hbm, zg_hbm, za_hbm, ones_hbm,
          g_out, a_out, c_out,
          src_v, dst_v, xrows_v, ea_v, ones_v, g_sh, a_sh, c_sh,
          sem_i, sem_g, sem_s, sem_e):
        c = lax.axis_index("c")
        s = lax.axis_index("s")
        r0 = s * rows_per_tile
        src_off = c * (n_groups * _GRP * _CH)

        pltpu.sync_copy(zg_hbm.at[pl.ds(r0, rows_per_tile)],
                        g_sh.at[pl.ds(r0, rows_per_tile)])
        pltpu.sync_copy(za_hbm.at[pl.ds(r0, rows_per_tile)],
                        a_sh.at[pl.ds(r0, rows_per_tile)])
        pltpu.sync_copy(za_hbm.at[pl.ds(r0, rows_per_tile)],
                        c_sh.at[pl.ds(r0, rows_per_tile)])
        pltpu.sync_copy(ones_hbm, ones_v)
        plsc.subcore_barrier()

        def body(kk, carry):
            gid = s + kk * _NS

            @pl.when(gid < n_groups)
            def _():
                base = gid * _GRP * _CH
                duty = (gid % _NC) == c
                pltpu.sync_copy(src2_hbm.at[pl.ds(src_off + base,
                                                  _GRP * _CH)], src_v)
                for j in range(_GRP):
                    pltpu.async_copy(dst_hbm.at[pl.ds(base + j * _CH, _CH)],
                                     dst_v.at[j], sem_i)

                @pl.when(duty)
                def _():
                    for j in range(_GRP):
                        pltpu.async_copy(
                            ea_hbm.at[pl.ds(base + j * _CH, _CH)],
                            ea_v.at[j], sem_e)

                for j in range(_GRP):
                    pltpu.async_copy(xs2_hbm.at[src_v.at[pl.ds(j * _CH,
                                                               _CH)]],
                                     xrows_v.at[j], sem_g)
                for j in range(_GRP):
                    pltpu.make_async_copy(dst_hbm.at[pl.ds(base + j * _CH,
                                                           _CH)],
                                          dst_v.at[j], sem_i).wait()
                for j in range(_GRP):
                    pltpu.make_async_copy(xs2_hbm.at[src_v.at[pl.ds(j * _CH,
                                                                    _CH)]],
                                          xrows_v.at[j], sem_g).wait()
                    pltpu.async_copy(xrows_v.at[j], g_sh.at[dst_v.at[j]],
                                     sem_s, add=True)

                @pl.when(duty)
                def _():
                    for j in range(_GRP):
                        pltpu.make_async_copy(
                            ea_hbm.at[pl.ds(base + j * _CH, _CH)],
                            ea_v.at[j], sem_e).wait()
                        pltpu.async_copy(ea_v.at[j], a_sh.at[dst_v.at[j]],
                                         sem_s, add=True)
                        pltpu.async_copy(ones_v, c_sh.at[dst_v.at[j]],
                                         sem_s, add=True)
                    for j in range(_GRP):
                        pltpu.make_async_copy(ea_v.at[j],
                                              a_sh.at[dst_v.at[j]],
                                              sem_s).wait()
                        pltpu.make_async_copy(ones_v,
                                              c_sh.at[dst_v.at[j]],
                                              sem_s).wait()

                for j in range(_GRP):
                    pltpu.make_async_copy(xrows_v.at[j],
                                          g_sh.at[dst_v.at[j]],
                                          sem_s).wait()

            return carry

        lax.fori_loop(0, iters, body, 0)
        plsc.subcore_barrier()

        pltpu.sync_copy(g_sh.at[pl.ds(r0, rows_per_tile)],
                        g_out.at[c, pl.ds(r0, rows_per_tile)])
        pltpu.sync_copy(a_sh.at[pl.ds(r0, rows_per_tile)],
                        a_out.at[c, pl.ds(r0, rows_per_tile)])
        pltpu.sync_copy(c_sh.at[pl.ds(r0, rows_per_tile)],
                        c_out.at[c, pl.ds(r0, rows_per_tile)])

    return k(xs2, src2, dst, ea, zg, za, ones_blk)


def _tc_dense(g2, a2, c2, xp, wx, we, bm, wn1, wn2, bn, w1, b1, w2, b2, w3, b3):
    n_pad, d = xp.shape
    dh = g2.shape[2]
    de = a2.shape[2]
    d_mlp = w3.shape[1]
    bn_rows = n_pad
    cand8 = n_pad // 8
    for cand in (1280, cand8 if cand8 % 8 == 0 else 0,
                 640, 320, 160, 128, 64, 32, 16, 8):
        if cand <= n_pad and n_pad % cand == 0:
            bn_rows = cand
            break
    grid = (n_pad // bn_rows,)

    def body(g_ref, a_ref, c_ref, x_ref, wx_ref, we_ref, bm_ref,
             wn1_ref, wn2_ref, bn_ref, w1_ref, b1_ref, w2_ref, b2_ref,
             w3_ref, b3_ref, o_ref):
        f32 = jnp.float32
        g = jnp.concatenate([g_ref[i] for i in range(_NC)], axis=-1)
        a = a_ref[0] + a_ref[1]
        cnt = (c_ref[0] + c_ref[1])[:, 0:1]
        summed = (jnp.dot(g, wx_ref[...], preferred_element_type=f32)
                  + jnp.dot(a, we_ref[...], preferred_element_type=f32)
                  + cnt * bm_ref[...])
        agg = summed / jnp.maximum(cnt, 1.0)
        h = (jnp.dot(x_ref[...], wn1_ref[...], preferred_element_type=f32)
             + jnp.dot(agg, wn2_ref[...], preferred_element_type=f32)
             + bn_ref[...])
        h = jnp.maximum(jnp.dot(h, w1_ref[...], preferred_element_type=f32)
                        + b1_ref[...], 0.0)
        h = jnp.maximum(jnp.dot(h, w2_ref[...], preferred_element_type=f32)
                        + b2_ref[...], 0.0)
        o_ref[...] = jnp.dot(h, w3_ref[...], preferred_element_type=f32) + b3_ref[...]

    full = lambda shape: pl.BlockSpec(shape, lambda i: (0,) * len(shape))
    return pl.pallas_call(
        body,
        grid=grid,
        in_specs=[
            pl.BlockSpec((_NC, bn_rows, dh), lambda i: (0, i, 0)),
            pl.BlockSpec((_NC, bn_rows, de), lambda i: (0, i, 0)),
            pl.BlockSpec((_NC, bn_rows, de), lambda i: (0, i, 0)),
            pl.BlockSpec((bn_rows, d), lambda i: (i, 0)),
            full(wx.shape), full(we.shape), full(bm.shape),
            full(wn1.shape), full(wn2.shape), full(bn.shape),
            full(w1.shape), full(b1.shape), full(w2.shape), full(b2.shape),
            full(w3.shape), full(b3.shape),
        ],
        out_specs=pl.BlockSpec((bn_rows, d_mlp), lambda i: (i, 0)),
        out_shape=jax.ShapeDtypeStruct((n_pad, d_mlp), jnp.float32),
    )(g2, a2, c2, xp, wx, we, bm, wn1, wn2, bn, w1, b1, w2, b2, w3, b3)


def kernel(x, edge_index, edge_attr, W_msg, b_msg, W_node, b_node,
           W1, b1, W2, b2, W3, b3):
    n, d = x.shape
    e = edge_index.shape[1]
    de = edge_attr.shape[1]

    rows_per_tile = -(-n // _NS)
    rows_per_tile = -(-rows_per_tile // 8) * 8
    n_pad = rows_per_tile * _NS

    src = edge_index[0].astype(jnp.int32)
    dst = edge_index[1].astype(jnp.int32)
    e_blk = _CH * _GRP
    e_pad = -(-e // e_blk) * e_blk
    if e_pad != e:
        src = jnp.concatenate([src, jnp.zeros((e_pad - e,), jnp.int32)])
        dst = jnp.concatenate([dst, jnp.full((e_pad - e,), n_pad - 1, jnp.int32)])
        edge_attr = jnp.concatenate(
            [edge_attr, jnp.zeros((e_pad - e, de), jnp.float32)])
    n_groups = e_pad // e_blk
    iters = -(-n_groups // _NS)

    dh = d // _NC
    xs2 = jnp.concatenate([x[:, c * dh:(c + 1) * dh] for c in range(_NC)])
    src2 = jnp.concatenate([src + c * n for c in range(_NC)])

    zg = jnp.zeros((n_pad, dh), jnp.float32)
    za = jnp.zeros((n_pad, de), jnp.float32)
    ones_blk = jnp.ones((_CH, de), jnp.float32)

    g2, a2, c2 = _sc_edge_aggregate(xs2, src2, dst, edge_attr, zg, za,
                                    ones_blk, n_pad, n_groups, iters)

    xp = jnp.concatenate([x, jnp.zeros((n_pad - n, d), jnp.float32)])
    wx = W_msg[:, :d].T
    we = W_msg[:, d:].T
    wn1 = W_node[:, :d].T
    wn2 = W_node[:, d:].T
    out = _tc_dense(g2, a2, c2, xp, wx, we, b_msg.reshape(1, -1),
                    wn1, wn2, b_node.reshape(1, -1),
                    W1.T, b1.reshape(1, -1), W2.T, b2.reshape(1, -1),
                    W3.T, b3.reshape(1, -1))
    return out[:n]
